# baseline (speedup 1.0000x reference)
import functools

import jax
import jax.numpy as jnp
from jax.experimental import pallas as pl

NUM_NEIGHBORS = 20
ROWS_PER_BLOCK = 256


def _knn_block_kernel(xr_ref, xca_ref, xcb_ref, out_ref, *, n_points, k):
    xr = xr_ref[0]
    xr0 = xr[:, 0:1]
    xr1 = xr[:, 1:2]
    xr2 = xr[:, 2:3]
    x2r = xr0 * xr0 + xr1 * xr1 + xr2 * xr2

    def _b(v):
        return v.astype(jnp.bfloat16).astype(jnp.float32)

    def _half_dist(xc):
        xc0 = xc[0:1, :]
        xc1 = xc[1:2, :]
        xc2 = xc[2:3, :]
        x2c = xc0 * xc0 + xc1 * xc1 + xc2 * xc2
        dot = _b(xr0) * _b(xc0) + _b(xr1) * _b(xc1) + _b(xr2) * _b(xc2)
        return (x2r + x2c) - 2.0 * dot

    d0 = _half_dist(xca_ref[0])
    d1 = _half_dist(xcb_ref[0])

    half = n_points // 2
    lane = jax.lax.broadcasted_iota(jnp.int32, d0.shape, 1)
    c0 = lane * 2
    c1 = c0 + 1
    take1 = d1 < d0
    e1 = jnp.where(take1, d1, d0)
    i1 = jnp.where(take1, c1, c0)
    e2 = jnp.where(take1, d0, d1)
    i2 = jnp.where(take1, c0, c1)

    inf = jnp.float32(jnp.inf)
    cols = []
    for _ in range(k):
        v, c = e1, i1
        w = half // 2
        while w >= 1:
            t = v[:, w:] < v[:, :w]
            v = jnp.where(t, v[:, w:], v[:, :w])
            c = jnp.where(t, c[:, w:], c[:, :w])
            w //= 2
        cols.append(c)
        mask = lane == jax.lax.div(c, 2)
        e1 = jnp.where(mask, e2, e1)
        i1 = jnp.where(mask, i2, i1)
        e2 = jnp.where(mask, inf, e2)
    out_ref[0] = jnp.concatenate(cols, axis=1)


def _knn_topk_indices(x):
    n_samples, n_points, _ = x.shape
    rows = ROWS_PER_BLOCK
    k = NUM_NEIGHBORS
    half = n_points // 2
    xr = jnp.pad(x, ((0, 0), (0, 0), (0, 1)))
    xct = jnp.swapaxes(x, 1, 2)
    xca = jnp.pad(xct[:, :, 0::2], ((0, 0), (0, 5), (0, 0)))
    xcb = jnp.pad(xct[:, :, 1::2], ((0, 0), (0, 5), (0, 0)))
    grid = (n_samples, n_points // rows)
    return pl.pallas_call(
        functools.partial(_knn_block_kernel, n_points=n_points, k=k),
        grid=grid,
        in_specs=[
            pl.BlockSpec((1, rows, 4), lambda s, r: (s, r, 0)),
            pl.BlockSpec((1, 8, half), lambda s, r: (s, 0, 0)),
            pl.BlockSpec((1, 8, half), lambda s, r: (s, 0, 0)),
        ],
        out_specs=pl.BlockSpec((1, rows, k), lambda s, r: (s, r, 0)),
        out_shape=jax.ShapeDtypeStruct((n_samples, n_points, k), jnp.int32),
    )(xr, xca, xcb)


def kernel(x):
    if x.ndim == 2:
        x = x[None, :, :]
    n_samples, n_points, _ = x.shape
    k_indices = _knn_topk_indices(x)
    dst = k_indices.astype(jnp.int64)
    src = jnp.zeros_like(dst) + jnp.arange(n_points, dtype=jnp.int64).reshape(1, -1, 1)
    per_sample_offset = (jnp.arange(n_samples, dtype=jnp.int64) * n_points).reshape(-1, 1, 1)
    dst = dst + per_sample_offset
    src = src + per_sample_offset
    return src.reshape(-1), dst.reshape(-1)


# speedup vs baseline: 2.3059x; 2.3059x over previous
import functools

import jax
import jax.numpy as jnp
from jax.experimental import pallas as pl

NUM_NEIGHBORS = 20
ROWS_PER_BLOCK = 256


def _knn_block_kernel(xr_ref, xca_ref, xcb_ref, out_ref, *, n_points, k):
    xr = xr_ref[0]
    xr0 = xr[:, 0:1]
    xr1 = xr[:, 1:2]
    xr2 = xr[:, 2:3]
    x2r = xr0 * xr0 + xr1 * xr1 + xr2 * xr2

    def _b(v):
        return v.astype(jnp.bfloat16).astype(jnp.float32)

    def _half_dist(xc):
        xc0 = xc[0:1, :]
        xc1 = xc[1:2, :]
        xc2 = xc[2:3, :]
        x2c = xc0 * xc0 + xc1 * xc1 + xc2 * xc2
        dot = _b(xr0) * _b(xc0) + _b(xr1) * _b(xc1) + _b(xr2) * _b(xc2)
        return (x2r + x2c) - 2.0 * dot

    d0 = _half_dist(xca_ref[0])
    d1 = _half_dist(xcb_ref[0])

    lane = jax.lax.broadcasted_iota(jnp.int32, d0.shape, 1)
    c0 = (lane * 2).astype(jnp.float32)
    c1 = c0 + 1.0
    take1 = d1 < d0
    e1 = jnp.where(take1, d1, d0)
    i1 = jnp.where(take1, c1, c0)
    e2 = jnp.where(take1, d0, d1)

    inf = jnp.float32(jnp.inf)
    big = jnp.float32(2 * n_points)
    cols = []
    for _ in range(k):
        m = jnp.min(e1, axis=1, keepdims=True)
        c = jnp.min(jnp.where(e1 == m, i1, big), axis=1, keepdims=True)
        cols.append(c)
        ci = c.astype(jnp.int32)
        mask = lane == jax.lax.shift_right_logical(ci, 1)
        e1 = jnp.where(mask, e2, e1)
        i1 = jnp.where(mask, jnp.bitwise_xor(ci, 1).astype(jnp.float32), i1)
        e2 = jnp.where(mask, inf, e2)
    out = jnp.concatenate(cols, axis=1)
    out_ref[0] = out.astype(jnp.int32)


def _knn_topk_indices(x):
    n_samples, n_points, _ = x.shape
    rows = ROWS_PER_BLOCK
    k = NUM_NEIGHBORS
    half = n_points // 2
    xr = jnp.pad(x, ((0, 0), (0, 0), (0, 1)))
    xct = jnp.swapaxes(x, 1, 2)
    xca = jnp.pad(xct[:, :, 0::2], ((0, 0), (0, 5), (0, 0)))
    xcb = jnp.pad(xct[:, :, 1::2], ((0, 0), (0, 5), (0, 0)))
    grid = (n_samples, n_points // rows)
    return pl.pallas_call(
        functools.partial(_knn_block_kernel, n_points=n_points, k=k),
        grid=grid,
        in_specs=[
            pl.BlockSpec((1, rows, 4), lambda s, r: (s, r, 0)),
            pl.BlockSpec((1, 8, half), lambda s, r: (s, 0, 0)),
            pl.BlockSpec((1, 8, half), lambda s, r: (s, 0, 0)),
        ],
        out_specs=pl.BlockSpec((1, rows, k), lambda s, r: (s, r, 0)),
        out_shape=jax.ShapeDtypeStruct((n_samples, n_points, k), jnp.int32),
    )(xr, xca, xcb)


def kernel(x):
    if x.ndim == 2:
        x = x[None, :, :]
    n_samples, n_points, _ = x.shape
    k_indices = _knn_topk_indices(x)
    dst = k_indices.astype(jnp.int64)
    src = jnp.zeros_like(dst) + jnp.arange(n_points, dtype=jnp.int64).reshape(1, -1, 1)
    per_sample_offset = (jnp.arange(n_samples, dtype=jnp.int64) * n_points).reshape(-1, 1, 1)
    dst = dst + per_sample_offset
    src = src + per_sample_offset
    return src.reshape(-1), dst.reshape(-1)


# speedup vs baseline: 2.3137x; 1.0034x over previous
import functools

import jax
import jax.numpy as jnp
from jax.experimental import pallas as pl

NUM_NEIGHBORS = 20
ROWS_PER_BLOCK = 512


def _knn_block_kernel(xr_ref, xca_ref, xcb_ref, out_ref, *, n_points, k):
    xr = xr_ref[0]
    xr0 = xr[:, 0:1]
    xr1 = xr[:, 1:2]
    xr2 = xr[:, 2:3]
    x2r = xr0 * xr0 + xr1 * xr1 + xr2 * xr2

    def _b(v):
        return v.astype(jnp.bfloat16).astype(jnp.float32)

    def _half_dist(xc):
        xc0 = xc[0:1, :]
        xc1 = xc[1:2, :]
        xc2 = xc[2:3, :]
        x2c = xc0 * xc0 + xc1 * xc1 + xc2 * xc2
        dot = _b(xr0) * _b(xc0) + _b(xr1) * _b(xc1) + _b(xr2) * _b(xc2)
        return (x2r + x2c) - 2.0 * dot

    d0 = _half_dist(xca_ref[0])
    d1 = _half_dist(xcb_ref[0])

    lane = jax.lax.broadcasted_iota(jnp.int32, d0.shape, 1)
    c0 = (lane * 2).astype(jnp.float32)
    c1 = c0 + 1.0
    take1 = d1 < d0
    e1 = jnp.where(take1, d1, d0)
    i1 = jnp.where(take1, c1, c0)
    e2 = jnp.where(take1, d0, d1)

    inf = jnp.float32(jnp.inf)
    big = jnp.float32(2 * n_points)
    cols = []
    for _ in range(k):
        m = jnp.min(e1, axis=1, keepdims=True)
        c = jnp.min(jnp.where(e1 == m, i1, big), axis=1, keepdims=True)
        cols.append(c)
        ci = c.astype(jnp.int32)
        mask = lane == jax.lax.shift_right_logical(ci, 1)
        e1 = jnp.where(mask, e2, e1)
        i1 = jnp.where(mask, jnp.bitwise_xor(ci, 1).astype(jnp.float32), i1)
        e2 = jnp.where(mask, inf, e2)
    out = jnp.concatenate(cols, axis=1)
    out_ref[0] = out.astype(jnp.int32)


def _knn_topk_indices(x):
    n_samples, n_points, _ = x.shape
    rows = ROWS_PER_BLOCK
    k = NUM_NEIGHBORS
    half = n_points // 2
    xr = jnp.pad(x, ((0, 0), (0, 0), (0, 1)))
    xct = jnp.swapaxes(x, 1, 2)
    xca = jnp.pad(xct[:, :, 0::2], ((0, 0), (0, 5), (0, 0)))
    xcb = jnp.pad(xct[:, :, 1::2], ((0, 0), (0, 5), (0, 0)))
    grid = (n_samples, n_points // rows)
    return pl.pallas_call(
        functools.partial(_knn_block_kernel, n_points=n_points, k=k),
        grid=grid,
        in_specs=[
            pl.BlockSpec((1, rows, 4), lambda s, r: (s, r, 0)),
            pl.BlockSpec((1, 8, half), lambda s, r: (s, 0, 0)),
            pl.BlockSpec((1, 8, half), lambda s, r: (s, 0, 0)),
        ],
        out_specs=pl.BlockSpec((1, rows, k), lambda s, r: (s, r, 0)),
        out_shape=jax.ShapeDtypeStruct((n_samples, n_points, k), jnp.int32),
    )(xr, xca, xcb)


def kernel(x):
    if x.ndim == 2:
        x = x[None, :, :]
    n_samples, n_points, _ = x.shape
    k_indices = _knn_topk_indices(x)
    dst = k_indices.astype(jnp.int64)
    src = jnp.zeros_like(dst) + jnp.arange(n_points, dtype=jnp.int64).reshape(1, -1, 1)
    per_sample_offset = (jnp.arange(n_samples, dtype=jnp.int64) * n_points).reshape(-1, 1, 1)
    dst = dst + per_sample_offset
    src = src + per_sample_offset
    return src.reshape(-1), dst.reshape(-1)
